# baseline (speedup 1.0000x reference)
import functools

import jax
import jax.numpy as jnp
from jax import lax
from jax.experimental import pallas as pl
from jax.experimental.pallas import tpu as pltpu
from jax.experimental.pallas import tpu_sc as plsc

_ROWS_PER_BLOCK = 256
_WAVE = 32


def _lse_block_kernel(x_ref, acc_ref):
    i = pl.program_id(0)
    x = x_ref[...]
    m = jnp.max(x, axis=1)
    s = jnp.sum(jnp.exp(x - m[:, None]), axis=1)
    lse = jnp.log(s) + m
    prev = jnp.where(i == 0, 0.0, acc_ref[0, 0])
    acc_ref[0, 0] = prev + jnp.sum(lse)


def _make_sc_gather(n, v):
    info = plsc.get_sparse_core_info()
    nw = info.num_cores * info.num_subcores
    lanes = info.num_lanes
    b_per_w = n // nw
    mesh = plsc.VectorSubcoreMesh(core_axis_name="c", subcore_axis_name="s")

    @functools.partial(
        pl.kernel,
        out_type=jax.ShapeDtypeStruct((nw, lanes), jnp.float32),
        mesh=mesh,
        compiler_params=pltpu.CompilerParams(needs_layout_passes=False),
        scratch_types=[
            pltpu.VMEM((b_per_w,), jnp.int32),
            pltpu.VMEM((b_per_w,), jnp.int32),
            pltpu.VMEM((b_per_w, lanes), jnp.float32),
            pltpu.VMEM((lanes,), jnp.float32),
            pltpu.SemaphoreType.DMA,
        ],
    )
    def sc_gather(logits_hbm, labels_hbm, out_hbm, lab_v, col_v,
                  rows_v, acc_v, sem):
        wid = lax.axis_index("s") * info.num_cores + lax.axis_index("c")
        base = wid * b_per_w
        pltpu.sync_copy(labels_hbm.at[pl.ds(base, b_per_w)], lab_v)
        lane_iota = lax.iota(jnp.int32, lanes)
        for j in range(b_per_w // lanes):
            lab = lab_v[pl.ds(j * lanes, lanes)]
            col_v[pl.ds(j * lanes, lanes)] = lax.bitwise_and(lab, lanes - 1)

        def row_start(i):
            grp = lab_v[pl.ds((i // lanes) * lanes, lanes)]
            sel = jnp.where(lane_iota == (i % lanes), grp, 0)
            lab_scalar = jnp.sum(sel)
            return (lab_scalar // lanes) * lanes

        def fire_wave(w):
            for k in range(_WAVE):
                i = w * _WAVE + k
                pltpu.async_copy(
                    logits_hbm.at[base + i, pl.ds(row_start(i), lanes)],
                    rows_v.at[i],
                    sem,
                )

        def drain_one():
            pltpu.make_async_copy(
                logits_hbm.at[base, pl.ds(0, lanes)], rows_v.at[0], sem
            ).wait()

        n_waves = b_per_w // _WAVE
        fire_wave(0)
        for w in range(1, n_waves):
            fire_wave(w)
            for _ in range(_WAVE):
                drain_one()
        for _ in range(_WAVE):
            drain_one()
        acc = jnp.zeros((lanes,), jnp.float32)
        for j in range(b_per_w // lanes):
            rows = j * lanes + lane_iota
            cols = col_v[pl.ds(j * lanes, lanes)]
            acc = acc + plsc.load_gather(rows_v, [rows, cols])
        acc_v[...] = acc
        pltpu.sync_copy(acc_v, out_hbm.at[wid])

    return sc_gather


def kernel(logits, labels):
    b, t, v = logits.shape
    n = b * t
    x = logits.reshape(n, v)
    lab = labels.reshape(n).astype(jnp.int32)

    partials = _make_sc_gather(n, v)(x, lab)

    r = _ROWS_PER_BLOCK
    g = n // r
    lse_sum = pl.pallas_call(
        _lse_block_kernel,
        grid=(g,),
        in_specs=[pl.BlockSpec((r, v), lambda i: (i, 0))],
        out_specs=pl.BlockSpec(
            (1, 1), lambda i: (0, 0), memory_space=pltpu.SMEM
        ),
        out_shape=jax.ShapeDtypeStruct((1, 1), jnp.float32),
        compiler_params=pltpu.CompilerParams(
            dimension_semantics=("arbitrary",),
            vmem_limit_bytes=128 * 1024 * 1024,
        ),
    )(x)

    return (lse_sum[0, 0] - jnp.sum(partials)) / n


# speedup vs baseline: 1.0055x; 1.0055x over previous
import functools

import jax
import jax.numpy as jnp
from jax import lax
from jax.experimental import pallas as pl
from jax.experimental.pallas import tpu as pltpu
from jax.experimental.pallas import tpu_sc as plsc

_ROWS_PER_BLOCK = 128
_WAVE = 32


def _lse_block_kernel(x_ref, acc_ref):
    i = pl.program_id(0)
    x = x_ref[...]
    m = jnp.max(x, axis=1)
    s = jnp.sum(jnp.exp(x - m[:, None]), axis=1)
    lse = jnp.log(s) + m
    prev = jnp.where(i == 0, 0.0, acc_ref[0, 0])
    acc_ref[0, 0] = prev + jnp.sum(lse)


def _make_sc_gather(n, v):
    info = plsc.get_sparse_core_info()
    nw = info.num_cores * info.num_subcores
    lanes = info.num_lanes
    b_per_w = n // nw
    mesh = plsc.VectorSubcoreMesh(core_axis_name="c", subcore_axis_name="s")

    @functools.partial(
        pl.kernel,
        out_type=jax.ShapeDtypeStruct((nw, lanes), jnp.float32),
        mesh=mesh,
        compiler_params=pltpu.CompilerParams(needs_layout_passes=False),
        scratch_types=[
            pltpu.VMEM((b_per_w,), jnp.int32),
            pltpu.VMEM((b_per_w,), jnp.int32),
            pltpu.VMEM((b_per_w, lanes), jnp.float32),
            pltpu.VMEM((lanes,), jnp.float32),
            pltpu.SemaphoreType.DMA,
        ],
    )
    def sc_gather(logits_hbm, labels_hbm, out_hbm, lab_v, col_v,
                  rows_v, acc_v, sem):
        wid = lax.axis_index("s") * info.num_cores + lax.axis_index("c")
        base = wid * b_per_w
        pltpu.sync_copy(labels_hbm.at[pl.ds(base, b_per_w)], lab_v)
        lane_iota = lax.iota(jnp.int32, lanes)
        for j in range(b_per_w // lanes):
            lab = lab_v[pl.ds(j * lanes, lanes)]
            col_v[pl.ds(j * lanes, lanes)] = lax.bitwise_and(lab, lanes - 1)

        def row_start(i):
            grp = lab_v[pl.ds((i // lanes) * lanes, lanes)]
            sel = jnp.where(lane_iota == (i % lanes), grp, 0)
            lab_scalar = jnp.sum(sel)
            return (lab_scalar // lanes) * lanes

        def fire_wave(w):
            for k in range(_WAVE):
                i = w * _WAVE + k
                pltpu.async_copy(
                    logits_hbm.at[base + i, pl.ds(row_start(i), lanes)],
                    rows_v.at[i],
                    sem,
                )

        def drain_one():
            pltpu.make_async_copy(
                logits_hbm.at[base, pl.ds(0, lanes)], rows_v.at[0], sem
            ).wait()

        n_waves = b_per_w // _WAVE
        fire_wave(0)
        for w in range(1, n_waves):
            fire_wave(w)
            for _ in range(_WAVE):
                drain_one()
        for _ in range(_WAVE):
            drain_one()
        acc = jnp.zeros((lanes,), jnp.float32)
        for j in range(b_per_w // lanes):
            rows = j * lanes + lane_iota
            cols = col_v[pl.ds(j * lanes, lanes)]
            acc = acc + plsc.load_gather(rows_v, [rows, cols])
        acc_v[...] = acc
        pltpu.sync_copy(acc_v, out_hbm.at[wid])

    return sc_gather


def kernel(logits, labels):
    b, t, v = logits.shape
    n = b * t
    x = logits.reshape(n, v)
    lab = labels.reshape(n).astype(jnp.int32)

    partials = _make_sc_gather(n, v)(x, lab)

    r = _ROWS_PER_BLOCK
    g = n // r
    lse_sum = pl.pallas_call(
        _lse_block_kernel,
        grid=(g,),
        in_specs=[pl.BlockSpec((r, v), lambda i: (i, 0))],
        out_specs=pl.BlockSpec(
            (1, 1), lambda i: (0, 0), memory_space=pltpu.SMEM
        ),
        out_shape=jax.ShapeDtypeStruct((1, 1), jnp.float32),
        compiler_params=pltpu.CompilerParams(
            dimension_semantics=("arbitrary",),
        ),
    )(x)

    return (lse_sum[0, 0] - jnp.sum(partials)) / n
